# baseline (speedup 1.0000x reference)
import functools

import jax
import jax.numpy as jnp
from jax import lax
from jax.experimental import pallas as pl
from jax.experimental.pallas import tpu as pltpu
from jax.experimental.pallas import tpu_sc as plsc

EMBED = 64
LANES = 16
NC = 2
NS = 16
NW = NC * NS

STEP = 400
PAIR = 2 * EMBED


@functools.cache
def _build(ntok):
    rows_per_w = ntok // NW
    nsteps = rows_per_w // STEP
    mesh = plsc.VectorSubcoreMesh(core_axis_name="c", subcore_axis_name="s")

    @functools.partial(
        pl.kernel,
        out_type=jax.ShapeDtypeStruct((ntok, PAIR), jnp.float32),
        mesh=mesh,
        compiler_params=pltpu.CompilerParams(use_tc_tiling_on_sc=False,
                                             needs_layout_passes=False),
        scratch_types=[
            pltpu.VMEM((rows_per_w,), jnp.int32),
            pltpu.VMEM((STEP, PAIR), jnp.float32),
            pltpu.VMEM((STEP, PAIR), jnp.float32),
            pltpu.SemaphoreType.DMA,
            pltpu.SemaphoreType.DMA,
            pltpu.SemaphoreType.DMA,
            pltpu.SemaphoreType.DMA,
        ],
    )
    def gather_kernel(tok_hbm, table_hbm, out_hbm,
                      tok_v, rows0, rows1, sgt0, sgt1, so0, so1):
        wid = lax.axis_index("s") * NC + lax.axis_index("c")
        w_base = wid * rows_per_w
        rows = (rows0, rows1)
        sgt = (sgt0, sgt1)
        so = (so0, so1)

        pltpu.async_copy(tok_hbm.at[pl.ds(w_base, rows_per_w)], tok_v, sgt0)
        pltpu.make_async_copy(tok_hbm.at[pl.ds(w_base, rows_per_w)], tok_v,
                              sgt0).wait()

        @plsc.parallel_loop(0, rows_per_w // LANES, unroll=8)
        def _halve(g):
            sl = pl.ds(g * LANES, LANES)
            tok_v[sl] = tok_v[sl] >> 1

        def issue_gather(st, b):
            off = pl.multiple_of(st * STEP, 8)
            pltpu.async_copy(table_hbm.at[tok_v.at[pl.ds(off, STEP)]],
                             rows[b], sgt[b])

        def wait_gather(b):
            pltpu.make_async_copy(table_hbm.at[tok_v.at[pl.ds(0, STEP)]],
                                  rows[b], sgt[b]).wait()

        def wait_out(b):
            pltpu.make_async_copy(rows[b], out_hbm.at[pl.ds(0, STEP)],
                                  so[b]).wait()

        issue_gather(0, 0)

        def pair_body(j, carry):
            for b in range(2):
                st = 2 * j + b
                nb = 1 - b

                @pl.when(st + 1 < nsteps)
                def _issue_next():
                    @pl.when(st >= 1)
                    def _drain_out():
                        wait_out(nb)
                    issue_gather(st + 1, nb)

                wait_gather(b)
                base = pl.multiple_of(w_base + st * STEP, 8)
                pltpu.async_copy(rows[b], out_hbm.at[pl.ds(base, STEP)], so[b])
            return carry

        lax.fori_loop(0, nsteps // 2, pair_body, 0)
        wait_out(0)
        wait_out(1)

    return gather_kernel


def kernel(x, padding_mask, table, pos_enc):
    b, s = x.shape
    ntok = b * s
    xf = x.reshape(ntok).astype(jnp.int32)
    table_pairs = table.reshape(table.shape[0] // 2, PAIR)
    g128 = _build(ntok)(xf, table_pairs)
    parity = (xf & 1).astype(jnp.bool_).reshape(b, s)
    lo = g128[:, :EMBED].reshape(b, s, EMBED)
    hi = g128[:, EMBED:].reshape(b, s, EMBED)
    g = jnp.where(parity[..., None], hi, lo)
    pos = pos_enc[None, :s, :].astype(jnp.float32)
    return g + jnp.where(padding_mask[..., None], 0.0, pos)


# speedup vs baseline: 1.4376x; 1.4376x over previous
import functools

import jax
import jax.numpy as jnp
from jax import lax
from jax.experimental import pallas as pl
from jax.experimental.pallas import tpu as pltpu
from jax.experimental.pallas import tpu_sc as plsc

EMBED = 64
NC = 2
NS = 16
NW = NC * NS

STEP = 800
HALVES = 1


@functools.cache
def _build(ntok):
    rows_per_w = ntok // NW
    nsteps = rows_per_w // STEP
    mesh = plsc.VectorSubcoreMesh(core_axis_name="c", subcore_axis_name="s")

    @functools.partial(
        pl.kernel,
        out_type=jax.ShapeDtypeStruct((ntok, EMBED), jnp.float32),
        mesh=mesh,
        compiler_params=pltpu.CompilerParams(use_tc_tiling_on_sc=False,
                                             needs_layout_passes=False),
        scratch_types=[
            pltpu.VMEM((rows_per_w,), jnp.int32),
            pltpu.VMEM((STEP, EMBED), jnp.float32),
            pltpu.VMEM((STEP, EMBED), jnp.float32),
            pltpu.SemaphoreType.DMA,
            pltpu.SemaphoreType.DMA,
            pltpu.SemaphoreType.DMA,
            pltpu.SemaphoreType.DMA,
        ],
    )
    def gather_kernel(tok_hbm, table_hbm, out_hbm,
                      tok_v, rows0, rows1, sgt0, sgt1, so0, so1):
        wid = lax.axis_index("s") * NC + lax.axis_index("c")
        w_base = wid * rows_per_w
        rows = (rows0, rows1)
        sgt = (sgt0, sgt1)
        so = (so0, so1)

        pltpu.async_copy(tok_hbm.at[pl.ds(w_base, rows_per_w)], tok_v, sgt0)
        pltpu.make_async_copy(tok_hbm.at[pl.ds(w_base, rows_per_w)], tok_v,
                              sgt0).wait()

        def issue_gather(st, b):
            off = pl.multiple_of(st * STEP, 8)
            pltpu.async_copy(table_hbm.at[tok_v.at[pl.ds(off, STEP)]],
                             rows[b], sgt[b])

        def wait_gather(b):
            pltpu.make_async_copy(table_hbm.at[tok_v.at[pl.ds(0, STEP)]],
                                  rows[b], sgt[b]).wait()

        def wait_out(b):
            pltpu.make_async_copy(rows[b], out_hbm.at[pl.ds(0, STEP)],
                                  so[b]).wait()

        issue_gather(0, 0)

        def pair_body(j, carry):
            for b in range(2):
                st = 2 * j + b
                nb = 1 - b

                @pl.when(st + 1 < nsteps)
                def _issue_next():
                    @pl.when(st >= 1)
                    def _drain_out():
                        wait_out(nb)
                    issue_gather(st + 1, nb)

                wait_gather(b)
                base = pl.multiple_of(w_base + st * STEP, 8)
                pltpu.async_copy(rows[b], out_hbm.at[pl.ds(base, STEP)], so[b])
            return carry

        lax.fori_loop(0, nsteps // 2, pair_body, 0)
        wait_out(0)
        wait_out(1)

    return gather_kernel


def kernel(x, padding_mask, table, pos_enc):
    b, s = x.shape
    bh = b // HALVES
    ntok_h = bh * s
    gk = _build(ntok_h)
    pos = pos_enc[None, :s, :].astype(jnp.float32)
    outs = []
    for h in range(HALVES):
        xh = x[h * bh:(h + 1) * bh].reshape(ntok_h).astype(jnp.int32)
        mh = padding_mask[h * bh:(h + 1) * bh]
        g = gk(xh, table).reshape(bh, s, EMBED)
        outs.append(g + jnp.where(mh[..., None], 0.0, pos))
    if HALVES == 1:
        return outs[0]
    return jnp.concatenate(outs, axis=0)
